# baseline (speedup 1.0000x reference)
import functools

import jax
import jax.numpy as jnp
from jax import lax
from jax.experimental import pallas as pl
from jax.experimental.pallas import tpu as pltpu
from jax.experimental.pallas import tpu_sc as plsc

VOCAB = 1000000
EMBED_DIM = 64
BATCH = 16384

NUM_CORES = 2
NUM_SUBCORES = 16

W = 256
BPS = 4096 // W
TAIL0 = 999936
NB_MAIN = 32768 // W
NB_LAST = 16896 // W
NS_MAIN = 8
NS_LAST = 5
OUT_ROWS = BATCH + 128
NSLOT = 4

_mesh = plsc.VectorSubcoreMesh(core_axis_name="c", subcore_axis_name="s")


@functools.partial(
    pl.kernel,
    mesh=_mesh,
    out_type=jax.ShapeDtypeStruct((OUT_ROWS, 128), jnp.float32),
    scratch_types=[
        pltpu.VMEM((BATCH,), jnp.int32),
        pltpu.VMEM((BATCH,), jnp.int32),
        pltpu.VMEM((BATCH,), jnp.int32),
        pltpu.VMEM((2, EMBED_DIM, W), jnp.float32),
        pltpu.VMEM((EMBED_DIM, 64), jnp.float32),
        pltpu.VMEM((NSLOT, 16, 128), jnp.float32),
        pltpu.VMEM((NSLOT, 16), jnp.int32),
        pltpu.SemaphoreType.DMA((2,)),
        pltpu.SemaphoreType.DMA((NSLOT,)),
    ],
    compiler_params=pltpu.CompilerParams(
        use_tc_tiling_on_sc=True, needs_layout_passes=False),
)
def _stream_kernel(idx_hbm, embt_hbm, out_hbm, scr, mypk, blkpk, blk, tailbuf,
                   stage, rpidx, bsem, ssem):
    wid = lax.axis_index("s") * NUM_CORES + lax.axis_index("c")
    iota16 = lax.iota(jnp.int32, 16)
    dump = jnp.full((16,), BATCH, jnp.int32) + wid

    wcap = jnp.minimum(wid, 30)
    lo = wcap * 32768
    nb = jnp.where(wid < 30, NB_MAIN, jnp.where(wid == 30, NB_LAST, 2))
    ns = jnp.where(wid < 30, NS_MAIN, jnp.where(wid == 30, NS_LAST, 1))

    pltpu.sync_copy(idx_hbm, scr)

    def fire_block(b):
        return pltpu.async_copy(
            embt_hbm.at[:, pl.ds(lo + b * W, W)],
            blk.at[b % 2],
            bsem.at[b % 2],
        )

    fire_block(0)

    def bin_body(i, cntv):
        v = scr[pl.ds(i * 16, 16)]
        owner = jnp.minimum(v >> 15, 30)
        m = ((owner == wid) & (v < TAIL0)) | ((v >= TAIL0) & (wid == 31))
        e = ((v - lo) << 14) | (i * 16 + iota16)
        plsc.store_scatter(mypk, [cntv * 16 + iota16], e, mask=m)
        return cntv + m.astype(jnp.int32)

    cntv = lax.fori_loop(0, BATCH // 16, bin_body,
                         jnp.zeros((16,), jnp.int32))
    njv = lax.reduce_max(cntv, axes=(0,))

    fire_block(1)

    def super_body(s, gg):
        def sup_body(j, scntv):
            e = mypk[pl.ds(j * 16, 16)]
            m = (cntv > j) & ((e >> 26) == s)
            plsc.store_scatter(scr, [scntv * 16 + iota16], e, mask=m)
            return scntv + m.astype(jnp.int32)

        scntv = lax.fori_loop(0, njv, sup_body, jnp.zeros((16,), jnp.int32))
        nsv = lax.reduce_max(scntv, axes=(0,))

        def block_body(bb, gg):
            b = s * BPS + bb

            pltpu.make_async_copy(
                embt_hbm.at[:, pl.ds(lo + b * W, W)], blk.at[b % 2],
                bsem.at[b % 2]
            ).wait()

            def scan_body(j, bcntv):
                e = scr[pl.ds(j * 16, 16)]
                m = (scntv > j) & ((e >> 22) == b)
                mi = m.astype(jnp.int32)
                r = bcntv + plsc.cumsum(mi) - mi
                plsc.store_scatter(blkpk, [r], e, mask=m)
                return bcntv + plsc.all_reduce_population_count(m)

            bcntv = lax.fori_loop(0, nsv, scan_body,
                                  jnp.zeros((16,), jnp.int32))
            ng = (lax.reduce_max(bcntv, axes=(0,)) + 15) >> 4

            def group_body(g, gg):
                slot = gg % NSLOT

                @pl.when(gg >= NSLOT)
                def _():
                    pltpu.make_async_copy(
                        stage.at[slot], out_hbm.at[rpidx.at[slot]],
                        ssem.at[slot]
                    ).wait()

                e = blkpk[pl.ds(g * 16, 16)]
                lmask = (g * 16 + iota16) < bcntv
                jc = (e >> 14) & (W - 1)
                rp = jnp.where(lmask, e & 16383, dump)
                rpidx[slot, :] = rp
                for d in range(EMBED_DIM):
                    dvec = jnp.full((16,), d, jnp.int32)
                    x = plsc.load_gather(blk.at[b % 2], [dvec, jc])
                    plsc.store_scatter(stage.at[slot], [iota16, dvec], x)
                pltpu.async_copy(
                    stage.at[slot], out_hbm.at[rpidx.at[slot]], ssem.at[slot]
                )
                return gg + 1

            gg = lax.fori_loop(0, ng, group_body, gg)

            @pl.when(b + 2 < nb)
            def _():
                fire_block(b + 2)

            return gg

        return lax.fori_loop(0, jnp.minimum(BPS, nb - s * BPS), block_body, gg)

    gg = lax.fori_loop(0, ns, super_body, jnp.int32(0))

    pltpu.sync_copy(embt_hbm.at[:, pl.ds(TAIL0, VOCAB - TAIL0)], tailbuf)

    def tail_body(j, gg):
        slot = gg % NSLOT

        @pl.when(gg >= NSLOT)
        def _():
            pltpu.make_async_copy(
                stage.at[slot], out_hbm.at[rpidx.at[slot]], ssem.at[slot]
            ).wait()

        e = mypk[pl.ds(j * 16, 16)]
        m = cntv > j
        jc = ((e >> 14) - (TAIL0 - 983040)) & 63
        rp = jnp.where(m, e & 16383, dump)
        rpidx[slot, :] = rp
        for d in range(EMBED_DIM):
            dvec = jnp.full((16,), d, jnp.int32)
            x = plsc.load_gather(tailbuf, [dvec, jc])
            plsc.store_scatter(stage.at[slot], [iota16, dvec], x)
        pltpu.async_copy(
            stage.at[slot], out_hbm.at[rpidx.at[slot]], ssem.at[slot]
        )
        return gg + 1

    gg = lax.fori_loop(0, jnp.where(wid == 31, njv, 0), tail_body, gg)

    def drain_body(g, x):
        slot = g % NSLOT
        pltpu.make_async_copy(
            stage.at[slot], out_hbm.at[rpidx.at[slot]], ssem.at[slot]
        ).wait()
        return x

    lax.fori_loop(jnp.maximum(gg - NSLOT, 0), gg, drain_body, jnp.int32(0))


def kernel(indices, embeddings):
    idx32 = indices.astype(jnp.int32)
    out_pad = _stream_kernel(idx32, embeddings.T)
    return out_pad[:BATCH, :EMBED_DIM]


# speedup vs baseline: 1.5196x; 1.5196x over previous
import functools

import jax
import jax.numpy as jnp
from jax import lax
from jax.experimental import pallas as pl
from jax.experimental.pallas import tpu as pltpu
from jax.experimental.pallas import tpu_sc as plsc

VOCAB = 1000000
EMBED_DIM = 64
BATCH = 16384

NUM_CORES = 2
NUM_SUBCORES = 16

W = 512
BPS = 4096 // W
TAIL0 = 999936
NB_MAIN = 32768 // W
NB_LAST = 16896 // W
NS_MAIN = 8
NS_LAST = 5
OUT_ROWS = BATCH + 128
NSLOT = 2

_mesh = plsc.VectorSubcoreMesh(core_axis_name="c", subcore_axis_name="s")


@functools.partial(
    pl.kernel,
    mesh=_mesh,
    out_type=jax.ShapeDtypeStruct((OUT_ROWS, 128), jnp.float32),
    scratch_types=[
        pltpu.VMEM((BATCH,), jnp.int32),
        pltpu.VMEM((BATCH,), jnp.int32),
        pltpu.VMEM((BATCH,), jnp.int32),
        pltpu.VMEM((2, EMBED_DIM, W), jnp.float32),
        pltpu.VMEM((EMBED_DIM, 64), jnp.float32),
        pltpu.VMEM((NSLOT, 16, 128), jnp.float32),
        pltpu.VMEM((NSLOT, 16), jnp.int32),
        pltpu.SemaphoreType.DMA((2,)),
        pltpu.SemaphoreType.DMA((NSLOT,)),
    ],
    compiler_params=pltpu.CompilerParams(
        use_tc_tiling_on_sc=True, needs_layout_passes=False),
)
def _stream_kernel(idx_hbm, embt_hbm, out_hbm, scr, mypk, blkpk, blk, tailbuf,
                   stage, rpidx, bsem, ssem):
    wid = lax.axis_index("s") * NUM_CORES + lax.axis_index("c")
    iota16 = lax.iota(jnp.int32, 16)
    dump = jnp.full((16,), BATCH, jnp.int32) + wid

    wcap = jnp.minimum(wid, 30)
    lo = wcap * 32768
    nb = jnp.where(wid < 30, NB_MAIN, jnp.where(wid == 30, NB_LAST, 2))
    ns = jnp.where(wid < 30, NS_MAIN, jnp.where(wid == 30, NS_LAST, 1))

    pltpu.sync_copy(idx_hbm, scr)

    def fire_block(b):
        return pltpu.async_copy(
            embt_hbm.at[:, pl.ds(lo + b * W, W)],
            blk.at[b % 2],
            bsem.at[b % 2],
        )

    fire_block(0)

    def bin_body(i, cntv):
        v = scr[pl.ds(i * 16, 16)]
        owner = jnp.minimum(v >> 15, 30)
        m = ((owner == wid) & (v < TAIL0)) | ((v >= TAIL0) & (wid == 31))
        e = ((v - lo) << 14) | (i * 16 + iota16)
        plsc.store_scatter(mypk, [cntv * 16 + iota16], e, mask=m)
        return cntv + m.astype(jnp.int32)

    cntv = lax.fori_loop(0, BATCH // 16, bin_body,
                         jnp.zeros((16,), jnp.int32))
    njv = lax.reduce_max(cntv, axes=(0,))

    fire_block(1)

    def super_body(s, gg):
        def sup_body(j, scntv):
            e = mypk[pl.ds(j * 16, 16)]
            m = (cntv > j) & ((e >> 26) == s)
            plsc.store_scatter(scr, [scntv * 16 + iota16], e, mask=m)
            return scntv + m.astype(jnp.int32)

        scntv = lax.fori_loop(0, njv, sup_body, jnp.zeros((16,), jnp.int32))
        nsv = lax.reduce_max(scntv, axes=(0,))

        def block_body(bb, gg):
            b = s * BPS + bb

            pltpu.make_async_copy(
                embt_hbm.at[:, pl.ds(lo + b * W, W)], blk.at[b % 2],
                bsem.at[b % 2]
            ).wait()

            def scan_body(j, bcntv):
                e = scr[pl.ds(j * 16, 16)]
                m = (scntv > j) & ((e >> 23) == b)
                mi = m.astype(jnp.int32)
                r = bcntv + plsc.cumsum(mi) - mi
                plsc.store_scatter(blkpk, [r], e, mask=m)
                return bcntv + plsc.all_reduce_population_count(m)

            bcntv = lax.fori_loop(0, nsv, scan_body,
                                  jnp.zeros((16,), jnp.int32))
            ng = (lax.reduce_max(bcntv, axes=(0,)) + 15) >> 4

            def group_body(g, gg):
                slot = gg % NSLOT

                @pl.when(gg >= NSLOT)
                def _():
                    pltpu.make_async_copy(
                        stage.at[slot], out_hbm.at[rpidx.at[slot]],
                        ssem.at[slot]
                    ).wait()

                e = blkpk[pl.ds(g * 16, 16)]
                lmask = (g * 16 + iota16) < bcntv
                jc = (e >> 14) & (W - 1)
                rp = jnp.where(lmask, e & 16383, dump)
                rpidx[slot, :] = rp
                for d in range(EMBED_DIM):
                    dvec = jnp.full((16,), d, jnp.int32)
                    x = plsc.load_gather(blk.at[b % 2], [dvec, jc])
                    plsc.store_scatter(stage.at[slot], [iota16, dvec], x)
                pltpu.async_copy(
                    stage.at[slot], out_hbm.at[rpidx.at[slot]], ssem.at[slot]
                )
                return gg + 1

            gg = lax.fori_loop(0, ng, group_body, gg)

            @pl.when(b + 2 < nb)
            def _():
                fire_block(b + 2)

            return gg

        return lax.fori_loop(0, jnp.minimum(BPS, nb - s * BPS), block_body, gg)

    gg = lax.fori_loop(0, ns, super_body, jnp.int32(0))

    pltpu.sync_copy(embt_hbm.at[:, pl.ds(TAIL0, VOCAB - TAIL0)], tailbuf)

    def tail_body(j, gg):
        slot = gg % NSLOT

        @pl.when(gg >= NSLOT)
        def _():
            pltpu.make_async_copy(
                stage.at[slot], out_hbm.at[rpidx.at[slot]], ssem.at[slot]
            ).wait()

        e = mypk[pl.ds(j * 16, 16)]
        m = cntv > j
        jc = ((e >> 14) - (TAIL0 - 983040)) & 63
        rp = jnp.where(m, e & 16383, dump)
        rpidx[slot, :] = rp
        for d in range(EMBED_DIM):
            dvec = jnp.full((16,), d, jnp.int32)
            x = plsc.load_gather(tailbuf, [dvec, jc])
            plsc.store_scatter(stage.at[slot], [iota16, dvec], x)
        pltpu.async_copy(
            stage.at[slot], out_hbm.at[rpidx.at[slot]], ssem.at[slot]
        )
        return gg + 1

    gg = lax.fori_loop(0, jnp.where(wid == 31, njv, 0), tail_body, gg)

    def drain_body(g, x):
        slot = g % NSLOT
        pltpu.make_async_copy(
            stage.at[slot], out_hbm.at[rpidx.at[slot]], ssem.at[slot]
        ).wait()
        return x

    lax.fori_loop(jnp.maximum(gg - NSLOT, 0), gg, drain_body, jnp.int32(0))


def kernel(indices, embeddings):
    idx32 = indices.astype(jnp.int32)
    out_pad = _stream_kernel(idx32, embeddings.T)
    return out_pad[:BATCH, :EMBED_DIM]
